# Initial kernel scaffold; baseline (speedup 1.0000x reference)
import jax
import jax.numpy as jnp
from jax.experimental import pallas as pl


def kernel(x, edge_index, batch, emb, W1_l, b1_l, W1_r, W2_l, b2_l, W2_r, W_lin, b_lin):
    raise NotImplementedError("write your pallas kernel here")


# speedup vs baseline: 4.8379x; 4.8379x over previous
import functools

import jax
import jax.numpy as jnp
from jax import lax
from jax.experimental import pallas as pl
from jax.experimental.pallas import tpu as pltpu
from jax.experimental.pallas import tpu_sc as plsc

N = 50000
E = 800000
V = 1000
D = 32
H = 64
G = 1024
NC = 2
NS = 16
NW = NC * NS

NP = 53248
EP = 802816
GA = 1152

@functools.cache
def _mesh():
  return plsc.VectorSubcoreMesh(core_axis_name="c", subcore_axis_name="s",
                                num_cores=NC, num_subcores=NS)


def _gather_rows(x_chunks, table):
  cw = NP // (NW * 128)

  @functools.partial(
      pl.kernel,
      out_type=jax.ShapeDtypeStruct((NP, D), jnp.float32),
      mesh=_mesh(),
      compiler_params=pltpu.CompilerParams(use_tc_tiling_on_sc=False,
                                           needs_layout_passes=False),
      scratch_types=[
          pltpu.VMEM((128,), jnp.int32),
          pltpu.VMEM((128, D), jnp.float32),
          pltpu.SemaphoreType.DMA,
      ],
  )
  def k(x_hbm, tab_hbm, out_hbm, idx_v, rows_v, sem):
    wid = lax.axis_index("s") * NC + lax.axis_index("c")

    def body(j, carry):
      crow = wid * cw + j
      pltpu.sync_copy(x_hbm.at[crow], idx_v)
      pltpu.async_copy(tab_hbm.at[idx_v], rows_v, sem).wait()
      pltpu.sync_copy(rows_v, out_hbm.at[pl.ds(crow * 128, 128)])
      return carry

    lax.fori_loop(0, cw, body, 0)

  return k(x_chunks, table)


def _make_segment_add(n_acc, cw):
  rt = n_acc // NS
  zc = rt if rt <= 256 else rt // 16
  nz = rt // zc

  @functools.partial(
      pl.kernel,
      out_type=(
          jax.ShapeDtypeStruct((NC, n_acc, D), jnp.float32),
          jax.ShapeDtypeStruct((NC, n_acc), jnp.float32),
      ),
      mesh=_mesh(),
      compiler_params=pltpu.CompilerParams(use_tc_tiling_on_sc=False,
                                           needs_layout_passes=False),
      scratch_types=[
          pltpu.VMEM_SHARED((n_acc, D), jnp.float32),
          pltpu.VMEM_SHARED((n_acc,), jnp.float32),
          pltpu.VMEM((zc, D), jnp.float32),
          pltpu.VMEM((rt,), jnp.float32),
          pltpu.VMEM((128,), jnp.int32),
          pltpu.VMEM((128,), jnp.int32),
          pltpu.VMEM((128, D), jnp.float32),
          pltpu.VMEM((128,), jnp.float32),
          pltpu.SemaphoreType.DMA,
      ],
  )
  def k(h_hbm, src_hbm, dst_hbm, zrows_hbm, zcnt_hbm, ones_hbm,
        out_rows, out_cnt,
        acc_sh, cnt_sh, zstage, cstage, idx_s, idx_d, rows_v, ones_v, sem):
    cid = lax.axis_index("c")
    sid = lax.axis_index("s")
    wid = sid * NC + cid

    pltpu.sync_copy(zrows_hbm, zstage)
    pltpu.sync_copy(zcnt_hbm, cstage)
    pltpu.sync_copy(ones_hbm, ones_v)
    for r in range(nz):
      pltpu.sync_copy(zstage, acc_sh.at[pl.ds(sid * rt + r * zc, zc)])
    pltpu.sync_copy(cstage, cnt_sh.at[pl.ds(sid * rt, rt)])
    plsc.subcore_barrier()

    def body(j, carry):
      crow = wid * cw + j
      pltpu.sync_copy(src_hbm.at[crow], idx_s)
      pltpu.sync_copy(dst_hbm.at[crow], idx_d)
      pltpu.async_copy(h_hbm.at[idx_s], rows_v, sem).wait()
      pltpu.sync_copy(rows_v, acc_sh.at[idx_d], add=True)
      pltpu.sync_copy(ones_v, cnt_sh.at[idx_d], add=True)
      return carry

    lax.fori_loop(0, cw, body, 0)
    plsc.subcore_barrier()

    for r in range(nz):
      off = sid * rt + r * zc
      pltpu.sync_copy(acc_sh.at[pl.ds(off, zc)], zstage)
      pltpu.sync_copy(zstage, out_rows.at[cid, pl.ds(off, zc)])
    pltpu.sync_copy(cnt_sh.at[pl.ds(sid * rt, rt)], cstage)
    pltpu.sync_copy(cstage, out_cnt.at[cid, pl.ds(sid * rt, rt)])

  return k


@functools.cache
def _seg_edges_kernel():
  return _make_segment_add(NP, EP // (NW * 128))


@functools.cache
def _seg_pool_kernel():
  return _make_segment_add(GA, NP // (NW * 128))


_TC_TILE = 2048
_TC_GRID = NP // _TC_TILE


def _layer1_tc(p, c, h0, wl_t, wr_t, b):
  def body(p_ref, c_ref, h_ref, wl_ref, wr_ref, b_ref, oa_ref, ob_ref):
    cnt = jnp.clip(jnp.sum(c_ref[...], axis=0), 1.0, None)
    agg = (p_ref[0] + p_ref[1]) / cnt[:, None]
    z = (jnp.dot(agg, wl_ref[...], preferred_element_type=jnp.float32)
         + b_ref[...]
         + jnp.dot(h_ref[...], wr_ref[...], preferred_element_type=jnp.float32))
    z = jnp.maximum(z, 0.0)
    oa_ref[...] = z[:, :D]
    ob_ref[...] = z[:, D:]

  return pl.pallas_call(
      body,
      grid=(_TC_GRID,),
      in_specs=[
          pl.BlockSpec((NC, _TC_TILE, D), lambda i: (0, i, 0)),
          pl.BlockSpec((NC, _TC_TILE), lambda i: (0, i)),
          pl.BlockSpec((_TC_TILE, D), lambda i: (i, 0)),
          pl.BlockSpec((D, H), lambda i: (0, 0)),
          pl.BlockSpec((D, H), lambda i: (0, 0)),
          pl.BlockSpec((1, H), lambda i: (0, 0)),
      ],
      out_specs=[
          pl.BlockSpec((_TC_TILE, D), lambda i: (i, 0)),
          pl.BlockSpec((_TC_TILE, D), lambda i: (i, 0)),
      ],
      out_shape=[
          jax.ShapeDtypeStruct((NP, D), jnp.float32),
          jax.ShapeDtypeStruct((NP, D), jnp.float32),
      ],
  )(p, c, h0, wl_t, wr_t, b)


def _layer2_tc(pa, pb, c, ha, hb, wl_t, wr_t, b):
  def body(pa_ref, pb_ref, c_ref, ha_ref, hb_ref, wl_ref, wr_ref, b_ref,
           oa_ref, ob_ref):
    cnt = jnp.clip(jnp.sum(c_ref[...], axis=0), 1.0, None)
    agg = jnp.concatenate(
        [pa_ref[0] + pa_ref[1], pb_ref[0] + pb_ref[1]], axis=1) / cnt[:, None]
    h = jnp.concatenate([ha_ref[...], hb_ref[...]], axis=1)
    z = (jnp.dot(agg, wl_ref[...], preferred_element_type=jnp.float32)
         + b_ref[...]
         + jnp.dot(h, wr_ref[...], preferred_element_type=jnp.float32))
    z = jnp.maximum(z, 0.0)
    oa_ref[...] = z[:, :D]
    ob_ref[...] = z[:, D:]

  return pl.pallas_call(
      body,
      grid=(_TC_GRID,),
      in_specs=[
          pl.BlockSpec((NC, _TC_TILE, D), lambda i: (0, i, 0)),
          pl.BlockSpec((NC, _TC_TILE, D), lambda i: (0, i, 0)),
          pl.BlockSpec((NC, _TC_TILE), lambda i: (0, i)),
          pl.BlockSpec((_TC_TILE, D), lambda i: (i, 0)),
          pl.BlockSpec((_TC_TILE, D), lambda i: (i, 0)),
          pl.BlockSpec((H, H), lambda i: (0, 0)),
          pl.BlockSpec((H, H), lambda i: (0, 0)),
          pl.BlockSpec((1, H), lambda i: (0, 0)),
      ],
      out_specs=[
          pl.BlockSpec((_TC_TILE, D), lambda i: (i, 0)),
          pl.BlockSpec((_TC_TILE, D), lambda i: (i, 0)),
      ],
      out_shape=[
          jax.ShapeDtypeStruct((NP, D), jnp.float32),
          jax.ShapeDtypeStruct((NP, D), jnp.float32),
      ],
  )(pa, pb, c, ha, hb, wl_t, wr_t, b)


def _final_tc(qa, qb, cg, wlin_t, blin):
  def body(qa_ref, qb_ref, cg_ref, w_ref, b_ref, o_ref):
    cnt = jnp.clip(jnp.sum(cg_ref[...], axis=0), 1.0, None)[:G]
    pooled = jnp.concatenate(
        [(qa_ref[0] + qa_ref[1])[:G], (qb_ref[0] + qb_ref[1])[:G]],
        axis=1) / cnt[:, None]
    o_ref[...] = (jnp.dot(pooled, w_ref[...],
                          preferred_element_type=jnp.float32) + b_ref[...])

  return pl.pallas_call(
      body,
      out_shape=jax.ShapeDtypeStruct((G, 16), jnp.float32),
  )(qa, qb, cg, wlin_t, blin)


def kernel(x, edge_index, batch, emb, W1_l, b1_l, W1_r, W2_l, b2_l, W2_r,
           W_lin, b_lin):
  x_flat = x.reshape(-1).astype(jnp.int32)
  x_pad = jnp.concatenate(
      [x_flat, jnp.zeros((NP - N,), jnp.int32)]).reshape(NP // 128, 128)
  src = edge_index[0].astype(jnp.int32)
  dst = edge_index[1].astype(jnp.int32)
  src_p = jnp.concatenate(
      [src, jnp.zeros((EP - E,), jnp.int32)]).reshape(EP // 128, 128)
  dst_p = jnp.concatenate(
      [dst, jnp.full((EP - E,), N, jnp.int32)]).reshape(EP // 128, 128)
  iota_src = jnp.concatenate(
      [jnp.arange(N, dtype=jnp.int32),
       jnp.zeros((NP - N,), jnp.int32)]).reshape(NP // 128, 128)
  batch_p = jnp.concatenate(
      [batch.astype(jnp.int32),
       jnp.full((NP - N,), G, jnp.int32)]).reshape(NP // 128, 128)

  z_rows_e = jnp.zeros((NP // NS // 16, D), jnp.float32)
  z_cnt_e = jnp.zeros((NP // NS,), jnp.float32)
  z_rows_g = jnp.zeros((GA // NS, D), jnp.float32)
  z_cnt_g = jnp.zeros((GA // NS,), jnp.float32)
  ones128 = jnp.ones((128,), jnp.float32)

  _seg_edges = _seg_edges_kernel()
  _seg_pool = _seg_pool_kernel()

  h0 = _gather_rows(x_pad, emb)

  p1, c1 = _seg_edges(h0, src_p, dst_p, z_rows_e, z_cnt_e, ones128)
  h1a, h1b = _layer1_tc(p1, c1, h0, W1_l.T, W1_r.T, b1_l.reshape(1, H))

  p2a, _ = _seg_edges(h1a, src_p, dst_p, z_rows_e, z_cnt_e, ones128)
  p2b, _ = _seg_edges(h1b, src_p, dst_p, z_rows_e, z_cnt_e, ones128)
  h2a, h2b = _layer2_tc(p2a, p2b, c1, h1a, h1b, W2_l.T, W2_r.T,
                        b2_l.reshape(1, H))

  qa, cg = _seg_pool(h2a, iota_src, batch_p, z_rows_g, z_cnt_g, ones128)
  qb, _ = _seg_pool(h2b, iota_src, batch_p, z_rows_g, z_cnt_g, ones128)

  wlin_t = jnp.pad(W_lin.T, ((0, 0), (0, 6)))
  blin = jnp.pad(b_lin, (0, 6)).reshape(1, 16)
  out16 = _final_tc(qa, qb, cg, wlin_t, blin)
  return out16[:, :10]


# speedup vs baseline: 7.2367x; 1.4958x over previous
import functools

import jax
import jax.numpy as jnp
from jax import lax
from jax.experimental import pallas as pl
from jax.experimental.pallas import tpu as pltpu
from jax.experimental.pallas import tpu_sc as plsc

N = 50000
E = 800000
V = 1000
D = 32
H = 64
G = 1024
NC = 2
NS = 16
NW = NC * NS

NP = 53248
EP = 802816
GA = 1152

@functools.cache
def _mesh():
  return plsc.VectorSubcoreMesh(core_axis_name="c", subcore_axis_name="s",
                                num_cores=NC, num_subcores=NS)


def _gather_rows(x_chunks, table):
  cw = NP // (NW * 128)

  @functools.partial(
      pl.kernel,
      out_type=jax.ShapeDtypeStruct((NP, D), jnp.float32),
      mesh=_mesh(),
      compiler_params=pltpu.CompilerParams(use_tc_tiling_on_sc=False,
                                           needs_layout_passes=False),
      scratch_types=[
          pltpu.VMEM((128,), jnp.int32),
          pltpu.VMEM((128, D), jnp.float32),
          pltpu.SemaphoreType.DMA,
      ],
  )
  def gather_rows_sc(x_hbm, tab_hbm, out_hbm, idx_v, rows_v, sem):
    wid = lax.axis_index("s") * NC + lax.axis_index("c")

    def body(j, carry):
      crow = wid * cw + j
      pltpu.sync_copy(x_hbm.at[crow], idx_v)
      pltpu.async_copy(tab_hbm.at[idx_v], rows_v, sem).wait()
      pltpu.sync_copy(rows_v, out_hbm.at[pl.ds(crow * 128, 128)])
      return carry

    lax.fori_loop(0, cw, body, 0)

  return gather_rows_sc(x_chunks, table)


def _make_segment_add(n_acc, cw):
  rt = n_acc // NS
  zc = rt if rt <= 256 else 128
  nz = rt // zc
  assert cw % 2 == 0

  @functools.partial(
      pl.kernel,
      out_type=(
          jax.ShapeDtypeStruct((NC, n_acc, D), jnp.float32),
          jax.ShapeDtypeStruct((NC, n_acc), jnp.float32),
      ),
      mesh=_mesh(),
      compiler_params=pltpu.CompilerParams(use_tc_tiling_on_sc=False,
                                           needs_layout_passes=False),
      scratch_types=[
          pltpu.VMEM_SHARED((n_acc, D), jnp.float32),
          pltpu.VMEM_SHARED((n_acc,), jnp.float32),
          pltpu.VMEM((zc, D), jnp.float32),
          pltpu.VMEM((rt,), jnp.float32),
          pltpu.VMEM((2, 2, 128), jnp.int32),
          pltpu.VMEM((128, D), jnp.float32),
          pltpu.VMEM((128, D), jnp.float32),
          pltpu.VMEM((128,), jnp.float32),
          pltpu.SemaphoreType.DMA,
          pltpu.SemaphoreType.DMA,
      ],
  )
  def segadd_sc(h_hbm, sdst_hbm, zrows_hbm, zcnt_hbm, ones_hbm,
        out_rows, out_cnt,
        acc_sh, cnt_sh, zstage, cstage, sd, rows0, rows1, ones_v,
        sem0, sem1):
    cid = lax.axis_index("c")
    sid = lax.axis_index("s")
    wid = sid * NC + cid
    rows = (rows0, rows1)
    sems = (sem0, sem1)

    pltpu.sync_copy(zrows_hbm, zstage)
    pltpu.sync_copy(zcnt_hbm, cstage)
    pltpu.sync_copy(ones_hbm, ones_v)
    for r in range(nz):
      pltpu.sync_copy(zstage, acc_sh.at[pl.ds(sid * rt + r * zc, zc)])
    pltpu.sync_copy(cstage, cnt_sh.at[pl.ds(sid * rt, rt)])
    plsc.subcore_barrier()

    base = wid * cw
    pltpu.sync_copy(sdst_hbm.at[base], sd.at[0])
    pltpu.async_copy(h_hbm.at[sd.at[0, 0]], rows[0], sems[0])

    def body(j, carry):
      for b in (0, 1):
        nxt = base + 2 * j + b + 1
        pltpu.sync_copy(sdst_hbm.at[nxt], sd.at[1 - b])
        pltpu.async_copy(h_hbm.at[sd.at[1 - b, 0]], rows[1 - b],
                         sems[1 - b])
        pltpu.make_async_copy(h_hbm.at[sd.at[b, 0]], rows[b],
                              sems[b]).wait()
        pltpu.sync_copy(rows[b], acc_sh.at[sd.at[b, 1]], add=True)
        pltpu.sync_copy(ones_v, cnt_sh.at[sd.at[b, 1]], add=True)
      return carry

    lax.fori_loop(0, cw // 2, body, 0)
    pltpu.make_async_copy(h_hbm.at[sd.at[0, 0]], rows[0], sems[0]).wait()
    plsc.subcore_barrier()

    for r in range(nz):
      off = sid * rt + r * zc
      pltpu.sync_copy(acc_sh.at[pl.ds(off, zc)], zstage)
      pltpu.sync_copy(zstage, out_rows.at[cid, pl.ds(off, zc)])
    pltpu.sync_copy(cnt_sh.at[pl.ds(sid * rt, rt)], cstage)
    pltpu.sync_copy(cstage, out_cnt.at[cid, pl.ds(sid * rt, rt)])

  return segadd_sc


@functools.cache
def _seg_edges_kernel():
  return _make_segment_add(NP, EP // (NW * 128))


PP = 57344


@functools.cache
def _seg_pool_kernel():
  return _make_segment_add(GA, PP // (NW * 128))


_TC_TILE = 2048
_TC_GRID = NP // _TC_TILE


def _layer1_tc(p, c, h0, wl_t, wr_t, b):
  def body(p_ref, c_ref, h_ref, wl_ref, wr_ref, b_ref, oa_ref, ob_ref):
    cnt = jnp.clip(jnp.sum(c_ref[...], axis=0), 1.0, None)
    agg = (p_ref[0] + p_ref[1]) / cnt[:, None]
    z = (jnp.dot(agg, wl_ref[...], preferred_element_type=jnp.float32)
         + b_ref[...]
         + jnp.dot(h_ref[...], wr_ref[...], preferred_element_type=jnp.float32))
    z = jnp.maximum(z, 0.0)
    oa_ref[...] = z[:, :D]
    ob_ref[...] = z[:, D:]

  return pl.pallas_call(
      body,
      grid=(_TC_GRID,),
      in_specs=[
          pl.BlockSpec((NC, _TC_TILE, D), lambda i: (0, i, 0)),
          pl.BlockSpec((NC, _TC_TILE), lambda i: (0, i)),
          pl.BlockSpec((_TC_TILE, D), lambda i: (i, 0)),
          pl.BlockSpec((D, H), lambda i: (0, 0)),
          pl.BlockSpec((D, H), lambda i: (0, 0)),
          pl.BlockSpec((1, H), lambda i: (0, 0)),
      ],
      out_specs=[
          pl.BlockSpec((_TC_TILE, D), lambda i: (i, 0)),
          pl.BlockSpec((_TC_TILE, D), lambda i: (i, 0)),
      ],
      out_shape=[
          jax.ShapeDtypeStruct((NP, D), jnp.float32),
          jax.ShapeDtypeStruct((NP, D), jnp.float32),
      ],
  )(p, c, h0, wl_t, wr_t, b)


def _layer2_tc(pa, pb, c, ha, hb, wl_t, wr_t, b):
  def body(pa_ref, pb_ref, c_ref, ha_ref, hb_ref, wl_ref, wr_ref, b_ref,
           oa_ref, ob_ref):
    cnt = jnp.clip(jnp.sum(c_ref[...], axis=0), 1.0, None)
    agg = jnp.concatenate(
        [pa_ref[0] + pa_ref[1], pb_ref[0] + pb_ref[1]], axis=1) / cnt[:, None]
    h = jnp.concatenate([ha_ref[...], hb_ref[...]], axis=1)
    z = (jnp.dot(agg, wl_ref[...], preferred_element_type=jnp.float32)
         + b_ref[...]
         + jnp.dot(h, wr_ref[...], preferred_element_type=jnp.float32))
    z = jnp.maximum(z, 0.0)
    oa_ref[...] = z[:, :D]
    ob_ref[...] = z[:, D:]

  return pl.pallas_call(
      body,
      grid=(_TC_GRID,),
      in_specs=[
          pl.BlockSpec((NC, _TC_TILE, D), lambda i: (0, i, 0)),
          pl.BlockSpec((NC, _TC_TILE, D), lambda i: (0, i, 0)),
          pl.BlockSpec((NC, _TC_TILE), lambda i: (0, i)),
          pl.BlockSpec((_TC_TILE, D), lambda i: (i, 0)),
          pl.BlockSpec((_TC_TILE, D), lambda i: (i, 0)),
          pl.BlockSpec((H, H), lambda i: (0, 0)),
          pl.BlockSpec((H, H), lambda i: (0, 0)),
          pl.BlockSpec((1, H), lambda i: (0, 0)),
      ],
      out_specs=[
          pl.BlockSpec((_TC_TILE, D), lambda i: (i, 0)),
          pl.BlockSpec((_TC_TILE, D), lambda i: (i, 0)),
      ],
      out_shape=[
          jax.ShapeDtypeStruct((NP, D), jnp.float32),
          jax.ShapeDtypeStruct((NP, D), jnp.float32),
      ],
  )(pa, pb, c, ha, hb, wl_t, wr_t, b)


def _final_tc(qa, qb, cg, wlin_t, blin):
  def body(qa_ref, qb_ref, cg_ref, w_ref, b_ref, o_ref):
    cnt = jnp.clip(jnp.sum(cg_ref[...], axis=0), 1.0, None)[:G]
    pooled = jnp.concatenate(
        [(qa_ref[0] + qa_ref[1])[:G], (qb_ref[0] + qb_ref[1])[:G]],
        axis=1) / cnt[:, None]
    o_ref[...] = (jnp.dot(pooled, w_ref[...],
                          preferred_element_type=jnp.float32) + b_ref[...])

  return pl.pallas_call(
      body,
      out_shape=jax.ShapeDtypeStruct((G, 16), jnp.float32),
  )(qa, qb, cg, wlin_t, blin)


def kernel(x, edge_index, batch, emb, W1_l, b1_l, W1_r, W2_l, b2_l, W2_r,
           W_lin, b_lin):
  x_flat = x.reshape(-1).astype(jnp.int32)
  x_pad = jnp.concatenate(
      [x_flat, jnp.zeros((NP - N,), jnp.int32)]).reshape(NP // 128, 128)
  src = edge_index[0].astype(jnp.int32)
  dst = edge_index[1].astype(jnp.int32)
  src_p = jnp.concatenate(
      [src, jnp.zeros((EP - E,), jnp.int32)]).reshape(EP // 128, 1, 128)
  dst_p = jnp.concatenate(
      [dst, jnp.full((EP - E,), N, jnp.int32)]).reshape(EP // 128, 1, 128)
  sdst_e = jnp.concatenate(
      [jnp.concatenate([src_p, dst_p], axis=1),
       jnp.zeros((1, 2, 128), jnp.int32)])
  iota_src = jnp.concatenate(
      [jnp.arange(N, dtype=jnp.int32),
       jnp.zeros((PP - N,), jnp.int32)]).reshape(PP // 128, 1, 128)
  batch_p = jnp.concatenate(
      [batch.astype(jnp.int32),
       jnp.full((PP - N,), G, jnp.int32)]).reshape(PP // 128, 1, 128)
  sdst_g = jnp.concatenate(
      [jnp.concatenate([iota_src, batch_p], axis=1),
       jnp.zeros((1, 2, 128), jnp.int32)])

  z_rows_e = jnp.zeros((128, D), jnp.float32)
  z_cnt_e = jnp.zeros((NP // NS,), jnp.float32)
  z_rows_g = jnp.zeros((GA // NS, D), jnp.float32)
  z_cnt_g = jnp.zeros((GA // NS,), jnp.float32)
  ones128 = jnp.ones((128,), jnp.float32)

  _seg_edges = _seg_edges_kernel()
  _seg_pool = _seg_pool_kernel()

  h0 = _gather_rows(x_pad, emb)

  p1, c1 = _seg_edges(h0, sdst_e, z_rows_e, z_cnt_e, ones128)
  h1a, h1b = _layer1_tc(p1, c1, h0, W1_l.T, W1_r.T, b1_l.reshape(1, H))

  p2a, _ = _seg_edges(h1a, sdst_e, z_rows_e, z_cnt_e, ones128)
  p2b, _ = _seg_edges(h1b, sdst_e, z_rows_e, z_cnt_e, ones128)
  h2a, h2b = _layer2_tc(p2a, p2b, c1, h1a, h1b, W2_l.T, W2_r.T,
                        b2_l.reshape(1, H))

  qa, cg = _seg_pool(h2a, sdst_g, z_rows_g, z_cnt_g, ones128)
  qb, _ = _seg_pool(h2b, sdst_g, z_rows_g, z_cnt_g, ones128)

  wlin_t = jnp.pad(W_lin.T, ((0, 0), (0, 6)))
  blin = jnp.pad(b_lin, (0, 6)).reshape(1, 16)
  out16 = _final_tc(qa, qb, cg, wlin_t, blin)
  return out16[:, :10]


# speedup vs baseline: 7.4034x; 1.0230x over previous
import functools

import jax
import jax.numpy as jnp
from jax import lax
from jax.experimental import pallas as pl
from jax.experimental.pallas import tpu as pltpu
from jax.experimental.pallas import tpu_sc as plsc

N = 50000
E = 800000
V = 1000
D = 32
H = 64
G = 1024
NC = 2
NS = 16
NW = NC * NS

NP = 53248
EP = 802816
GA = 1152

@functools.cache
def _mesh():
  return plsc.VectorSubcoreMesh(core_axis_name="c", subcore_axis_name="s",
                                num_cores=NC, num_subcores=NS)


def _gather_rows(x_chunks, table):
  cw = NP // (NW * 128)

  @functools.partial(
      pl.kernel,
      out_type=jax.ShapeDtypeStruct((NP, D), jnp.float32),
      mesh=_mesh(),
      compiler_params=pltpu.CompilerParams(use_tc_tiling_on_sc=False,
                                           needs_layout_passes=False),
      scratch_types=[
          pltpu.VMEM((128,), jnp.int32),
          pltpu.VMEM((128, D), jnp.float32),
          pltpu.SemaphoreType.DMA,
      ],
  )
  def gather_rows_sc(x_hbm, tab_hbm, out_hbm, idx_v, rows_v, sem):
    wid = lax.axis_index("s") * NC + lax.axis_index("c")

    def body(j, carry):
      crow = wid * cw + j
      pltpu.sync_copy(x_hbm.at[crow], idx_v)
      pltpu.async_copy(tab_hbm.at[idx_v], rows_v, sem).wait()
      pltpu.sync_copy(rows_v, out_hbm.at[pl.ds(crow * 128, 128)])
      return carry

    lax.fori_loop(0, cw, body, 0)

  return gather_rows_sc(x_chunks, table)


def _make_segment_add(n_acc, cw, with_cnt=True):
  rt = n_acc // NS
  zc = rt if rt <= 256 else 128
  nz = rt // zc
  assert cw % 2 == 0

  @functools.partial(
      pl.kernel,
      out_type=(
          jax.ShapeDtypeStruct((NC, n_acc, D), jnp.float32),
          jax.ShapeDtypeStruct((NC, n_acc), jnp.float32),
      ),
      mesh=_mesh(),
      compiler_params=pltpu.CompilerParams(use_tc_tiling_on_sc=False,
                                           needs_layout_passes=False),
      scratch_types=[
          pltpu.VMEM_SHARED((n_acc, D), jnp.float32),
          pltpu.VMEM_SHARED((n_acc,), jnp.float32),
          pltpu.VMEM((zc, D), jnp.float32),
          pltpu.VMEM((rt,), jnp.float32),
          pltpu.VMEM((2, 2, 128), jnp.int32),
          pltpu.VMEM((128, D), jnp.float32),
          pltpu.VMEM((128, D), jnp.float32),
          pltpu.VMEM((128,), jnp.float32),
          pltpu.SemaphoreType.DMA,
          pltpu.SemaphoreType.DMA,
      ],
  )
  def segadd_sc(h_hbm, sdst_hbm, zrows_hbm, zcnt_hbm, ones_hbm,
        out_rows, out_cnt,
        acc_sh, cnt_sh, zstage, cstage, sd, rows0, rows1, ones_v,
        sem0, sem1):
    cid = lax.axis_index("c")
    sid = lax.axis_index("s")
    wid = sid * NC + cid
    rows = (rows0, rows1)
    sems = (sem0, sem1)

    pltpu.sync_copy(zrows_hbm, zstage)
    if with_cnt:
      pltpu.sync_copy(zcnt_hbm, cstage)
      pltpu.sync_copy(ones_hbm, ones_v)
    for r in range(nz):
      pltpu.sync_copy(zstage, acc_sh.at[pl.ds(sid * rt + r * zc, zc)])
    if with_cnt:
      pltpu.sync_copy(cstage, cnt_sh.at[pl.ds(sid * rt, rt)])
    plsc.subcore_barrier()

    base = wid * cw
    pltpu.sync_copy(sdst_hbm.at[base], sd.at[0])
    pltpu.async_copy(h_hbm.at[sd.at[0, 0]], rows[0], sems[0])

    def body(j, carry):
      for b in (0, 1):
        nxt = base + 2 * j + b + 1
        pltpu.sync_copy(sdst_hbm.at[nxt], sd.at[1 - b])
        pltpu.async_copy(h_hbm.at[sd.at[1 - b, 0]], rows[1 - b],
                         sems[1 - b])
        pltpu.make_async_copy(h_hbm.at[sd.at[b, 0]], rows[b],
                              sems[b]).wait()
        pltpu.sync_copy(rows[b], acc_sh.at[sd.at[b, 1]], add=True)
        if with_cnt:
          pltpu.sync_copy(ones_v, cnt_sh.at[sd.at[b, 1]], add=True)
      return carry

    lax.fori_loop(0, cw // 2, body, 0)
    pltpu.make_async_copy(h_hbm.at[sd.at[0, 0]], rows[0], sems[0]).wait()
    plsc.subcore_barrier()

    for r in range(nz):
      off = sid * rt + r * zc
      pltpu.sync_copy(acc_sh.at[pl.ds(off, zc)], zstage)
      pltpu.sync_copy(zstage, out_rows.at[cid, pl.ds(off, zc)])
    if with_cnt:
      pltpu.sync_copy(cnt_sh.at[pl.ds(sid * rt, rt)], cstage)
      pltpu.sync_copy(cstage, out_cnt.at[cid, pl.ds(sid * rt, rt)])

  return segadd_sc


@functools.cache
def _seg_edges_kernel():
  return _make_segment_add(NP, EP // (NW * 128))


@functools.cache
def _seg_edges_nocnt_kernel():
  return _make_segment_add(NP, EP // (NW * 128), with_cnt=False)


PP = 57344


@functools.cache
def _seg_pool_kernel():
  return _make_segment_add(GA, PP // (NW * 128))


_TC_TILE = 2048
_TC_GRID = NP // _TC_TILE


def _layer1_tc(p, c, h0, wl_t, wr_t, b):
  def body(p_ref, c_ref, h_ref, wl_ref, wr_ref, b_ref, oa_ref, ob_ref):
    cnt = jnp.clip(jnp.sum(c_ref[...], axis=0), 1.0, None)
    agg = (p_ref[0] + p_ref[1]) / cnt[:, None]
    z = (jnp.dot(agg, wl_ref[...], preferred_element_type=jnp.float32)
         + b_ref[...]
         + jnp.dot(h_ref[...], wr_ref[...], preferred_element_type=jnp.float32))
    z = jnp.maximum(z, 0.0)
    oa_ref[...] = z[:, :D]
    ob_ref[...] = z[:, D:]

  return pl.pallas_call(
      body,
      grid=(_TC_GRID,),
      in_specs=[
          pl.BlockSpec((NC, _TC_TILE, D), lambda i: (0, i, 0)),
          pl.BlockSpec((NC, _TC_TILE), lambda i: (0, i)),
          pl.BlockSpec((_TC_TILE, D), lambda i: (i, 0)),
          pl.BlockSpec((D, H), lambda i: (0, 0)),
          pl.BlockSpec((D, H), lambda i: (0, 0)),
          pl.BlockSpec((1, H), lambda i: (0, 0)),
      ],
      out_specs=[
          pl.BlockSpec((_TC_TILE, D), lambda i: (i, 0)),
          pl.BlockSpec((_TC_TILE, D), lambda i: (i, 0)),
      ],
      out_shape=[
          jax.ShapeDtypeStruct((NP, D), jnp.float32),
          jax.ShapeDtypeStruct((NP, D), jnp.float32),
      ],
  )(p, c, h0, wl_t, wr_t, b)


def _layer2_tc(pa, pb, c, ha, hb, wl_t, wr_t, b):
  def body(pa_ref, pb_ref, c_ref, ha_ref, hb_ref, wl_ref, wr_ref, b_ref,
           oa_ref, ob_ref):
    cnt = jnp.clip(jnp.sum(c_ref[...], axis=0), 1.0, None)
    agg = jnp.concatenate(
        [pa_ref[0] + pa_ref[1], pb_ref[0] + pb_ref[1]], axis=1) / cnt[:, None]
    h = jnp.concatenate([ha_ref[...], hb_ref[...]], axis=1)
    z = (jnp.dot(agg, wl_ref[...], preferred_element_type=jnp.float32)
         + b_ref[...]
         + jnp.dot(h, wr_ref[...], preferred_element_type=jnp.float32))
    z = jnp.maximum(z, 0.0)
    oa_ref[...] = z[:, :D]
    ob_ref[...] = z[:, D:]

  return pl.pallas_call(
      body,
      grid=(_TC_GRID,),
      in_specs=[
          pl.BlockSpec((NC, _TC_TILE, D), lambda i: (0, i, 0)),
          pl.BlockSpec((NC, _TC_TILE, D), lambda i: (0, i, 0)),
          pl.BlockSpec((NC, _TC_TILE), lambda i: (0, i)),
          pl.BlockSpec((_TC_TILE, D), lambda i: (i, 0)),
          pl.BlockSpec((_TC_TILE, D), lambda i: (i, 0)),
          pl.BlockSpec((H, H), lambda i: (0, 0)),
          pl.BlockSpec((H, H), lambda i: (0, 0)),
          pl.BlockSpec((1, H), lambda i: (0, 0)),
      ],
      out_specs=[
          pl.BlockSpec((_TC_TILE, D), lambda i: (i, 0)),
          pl.BlockSpec((_TC_TILE, D), lambda i: (i, 0)),
      ],
      out_shape=[
          jax.ShapeDtypeStruct((NP, D), jnp.float32),
          jax.ShapeDtypeStruct((NP, D), jnp.float32),
      ],
  )(pa, pb, c, ha, hb, wl_t, wr_t, b)


def _final_tc(qa, qb, cg, wlin_t, blin):
  def body(qa_ref, qb_ref, cg_ref, w_ref, b_ref, o_ref):
    cnt = jnp.clip(jnp.sum(cg_ref[...], axis=0), 1.0, None)[:G]
    pooled = jnp.concatenate(
        [(qa_ref[0] + qa_ref[1])[:G], (qb_ref[0] + qb_ref[1])[:G]],
        axis=1) / cnt[:, None]
    o_ref[...] = (jnp.dot(pooled, w_ref[...],
                          preferred_element_type=jnp.float32) + b_ref[...])

  return pl.pallas_call(
      body,
      out_shape=jax.ShapeDtypeStruct((G, 16), jnp.float32),
  )(qa, qb, cg, wlin_t, blin)


def kernel(x, edge_index, batch, emb, W1_l, b1_l, W1_r, W2_l, b2_l, W2_r,
           W_lin, b_lin):
  x_flat = x.reshape(-1).astype(jnp.int32)
  x_pad = jnp.concatenate(
      [x_flat, jnp.zeros((NP - N,), jnp.int32)]).reshape(NP // 128, 128)
  src = edge_index[0].astype(jnp.int32)
  dst = edge_index[1].astype(jnp.int32)
  src_p = jnp.concatenate(
      [src, jnp.zeros((EP - E,), jnp.int32)]).reshape(EP // 128, 1, 128)
  dst_p = jnp.concatenate(
      [dst, jnp.full((EP - E,), N, jnp.int32)]).reshape(EP // 128, 1, 128)
  sdst_e = jnp.concatenate(
      [jnp.concatenate([src_p, dst_p], axis=1),
       jnp.zeros((1, 2, 128), jnp.int32)])
  iota_src = jnp.concatenate(
      [jnp.arange(N, dtype=jnp.int32),
       jnp.zeros((PP - N,), jnp.int32)]).reshape(PP // 128, 1, 128)
  batch_p = jnp.concatenate(
      [batch.astype(jnp.int32),
       jnp.full((PP - N,), G, jnp.int32)]).reshape(PP // 128, 1, 128)
  sdst_g = jnp.concatenate(
      [jnp.concatenate([iota_src, batch_p], axis=1),
       jnp.zeros((1, 2, 128), jnp.int32)])

  z_rows_e = jnp.zeros((128, D), jnp.float32)
  z_cnt_e = jnp.zeros((NP // NS,), jnp.float32)
  z_rows_g = jnp.zeros((GA // NS, D), jnp.float32)
  z_cnt_g = jnp.zeros((GA // NS,), jnp.float32)
  ones128 = jnp.ones((128,), jnp.float32)

  _seg_edges = _seg_edges_kernel()
  _seg_pool = _seg_pool_kernel()

  h0 = _gather_rows(x_pad, emb)

  p1, c1 = _seg_edges(h0, sdst_e, z_rows_e, z_cnt_e, ones128)
  h1a, h1b = _layer1_tc(p1, c1, h0, W1_l.T, W1_r.T, b1_l.reshape(1, H))

  _seg_edges_nc = _seg_edges_nocnt_kernel()
  p2a, _ = _seg_edges_nc(h1a, sdst_e, z_rows_e, z_cnt_e, ones128)
  p2b, _ = _seg_edges_nc(h1b, sdst_e, z_rows_e, z_cnt_e, ones128)
  h2a, h2b = _layer2_tc(p2a, p2b, c1, h1a, h1b, W2_l.T, W2_r.T,
                        b2_l.reshape(1, H))

  qa, cg = _seg_pool(h2a, sdst_g, z_rows_g, z_cnt_g, ones128)
  qb, _ = _seg_pool(h2b, sdst_g, z_rows_g, z_cnt_g, ones128)

  wlin_t = jnp.pad(W_lin.T, ((0, 0), (0, 6)))
  blin = jnp.pad(b_lin, (0, 6)).reshape(1, 16)
  out16 = _final_tc(qa, qb, cg, wlin_t, blin)
  return out16[:, :10]
